# baseline (speedup 1.0000x reference)
import functools

import jax
import jax.numpy as jnp
from jax import lax
from jax.experimental import pallas as pl
from jax.experimental.pallas import tpu as pltpu
from jax.experimental.pallas import tpu_sc as plsc

N = 100000
B = 100
D = 128
L = 6
R = 6
V = 50000
GMAX = 1200
CMAX = 512
EPS = 1e-5

NBLK = 1024
NBC = 98
NPAD = NBLK * NBC
KE = 1280
BP = 128
ZR = NPAD


@functools.lru_cache(maxsize=None)
def _make_sc_gather(n_rows, d, dtype_name):
    dtype = jnp.dtype(dtype_name)
    info = plsc.get_sparse_core_info()
    nw = info.num_cores * info.num_subcores
    assert n_rows % (8 * nw) == 0
    bpw = n_rows // nw
    esz = dtype.itemsize
    chunk = 8
    for c in range(8, bpw + 1, 8):
        if bpw % c == 0 and c * (d * esz + 4) <= 420000:
            chunk = c
    n_iter = bpw // chunk
    mesh = plsc.VectorSubcoreMesh(core_axis_name="c", subcore_axis_name="s")

    @functools.partial(
        pl.kernel,
        mesh=mesh,
        out_type=jax.ShapeDtypeStruct((n_rows, d), dtype),
        scratch_types=[
            pltpu.VMEM((chunk,), jnp.int32),
            pltpu.VMEM((chunk, d), dtype),
            pltpu.SemaphoreType.DMA,
        ],
    )
    def gk(table_hbm, idx_hbm, out_hbm, idx_v, rows_v, sem):
        wid = lax.axis_index("s") * info.num_cores + lax.axis_index("c")
        base = wid * bpw

        def body(j, carry):
            off = base + j * chunk
            pltpu.sync_copy(idx_hbm.at[pl.ds(off, chunk)], idx_v)
            pltpu.async_copy(table_hbm.at[idx_v], rows_v, sem).wait()
            pltpu.sync_copy(rows_v, out_hbm.at[pl.ds(off, chunk)])
            return carry

        lax.fori_loop(0, n_iter, body, 0)

    return gk


def _sc_gather(table, idx):
    n = idx.shape[0]
    np_ = (-n) % 256
    if np_:
        idx = jnp.concatenate([idx, jnp.zeros((np_,), jnp.int32)])
    out = _make_sc_gather(n + np_, table.shape[1], str(table.dtype))(table, idx)
    return out[:n]


def _layer_a(msgs_ref, loc_ref, h_ref, bat_ref, w_ref, wr_ref, bs_ref,
             hpre_ref, stats_ref):
    i = pl.program_id(0)
    h = h_ref[...]
    acc = jax.lax.dot_general(h, wr_ref[...], (((1,), (0,)), ((), ())),
                              preferred_element_type=jnp.float32, precision=lax.Precision.HIGHEST)
    acc = acc + bs_ref[...]
    iota_n = lax.broadcasted_iota(jnp.int32, (1, NBLK), 1)
    for r in range(R):
        msg = msgs_ref[0, r]
        lc = loc_ref[0, r]
        oh = (lc == iota_n).astype(jnp.float32)
        s = jax.lax.dot_general(oh, msg, (((0,), (0,)), ((), ())),
                                preferred_element_type=jnp.float32, precision=lax.Precision.HIGHEST)
        deg = jnp.sum(oh, axis=0)[:, None]
        mean = s / jnp.maximum(deg, 1.0)
        acc = acc + jax.lax.dot_general(mean, w_ref[r], (((1,), (0,)), ((), ())),
                                        preferred_element_type=jnp.float32, precision=lax.Precision.HIGHEST)
    hpre = h + jnp.maximum(acc, 0.0)
    hpre_ref[...] = hpre
    iota_b = lax.broadcasted_iota(jnp.int32, (1, BP), 1)
    ohb = (bat_ref[...] == iota_b).astype(jnp.float32)
    st = jax.lax.dot_general(ohb, jnp.concatenate([hpre, hpre * hpre], axis=1),
                             (((0,), (0,)), ((), ())),
                             preferred_element_type=jnp.float32, precision=lax.Precision.HIGHEST)

    @pl.when(i == 0)
    def _():
        stats_ref[...] = st

    @pl.when(i != 0)
    def _():
        stats_ref[...] = stats_ref[...] + st


def _layer_b(hpre_ref, bat_ref, stats_ref, cnt_ref, g_ref, b_ref, a_ref,
             out_ref):
    st = stats_ref[...]
    cnt = cnt_ref[...]
    m = st[:, :D] / cnt
    e2 = st[:, D:] / cnt
    a = a_ref[...]
    var = e2 - (2.0 * a - a * a) * (m * m)
    inv = 1.0 / jnp.sqrt(var + EPS)
    iota_b = lax.broadcasted_iota(jnp.int32, (1, BP), 1)
    ohb = (bat_ref[...] == iota_b).astype(jnp.float32)
    mi = jax.lax.dot_general(ohb, jnp.concatenate([m, inv], axis=1),
                             (((1,), (0,)), ((), ())),
                             preferred_element_type=jnp.float32, precision=lax.Precision.HIGHEST)
    hp = hpre_ref[...]
    out_ref[...] = g_ref[...] * (hp - a * mi[:, :D]) * mi[:, D:] + b_ref[...]


def _run_layer(h, msgs, loc, bat2, cnt2, wl, wr, bsum, g, b, a):
    hpre, stats = pl.pallas_call(
        _layer_a,
        grid=(NBC,),
        in_specs=[
            pl.BlockSpec((1, R, KE, D), lambda i: (i, 0, 0, 0)),
            pl.BlockSpec((1, R, KE, 1), lambda i: (i, 0, 0, 0)),
            pl.BlockSpec((NBLK, D), lambda i: (i, 0)),
            pl.BlockSpec((NBLK, 1), lambda i: (i, 0)),
            pl.BlockSpec((R, D, D), lambda i: (0, 0, 0)),
            pl.BlockSpec((D, D), lambda i: (0, 0)),
            pl.BlockSpec((1, D), lambda i: (0, 0)),
        ],
        out_specs=[
            pl.BlockSpec((NBLK, D), lambda i: (i, 0)),
            pl.BlockSpec((BP, 2 * D), lambda i: (0, 0)),
        ],
        out_shape=[
            jax.ShapeDtypeStruct((NPAD, D), jnp.float32),
            jax.ShapeDtypeStruct((BP, 2 * D), jnp.float32),
        ],
    )(msgs, loc, h, bat2, wl, wr, bsum)
    h2 = pl.pallas_call(
        _layer_b,
        grid=(NBC,),
        in_specs=[
            pl.BlockSpec((NBLK, D), lambda i: (i, 0)),
            pl.BlockSpec((NBLK, 1), lambda i: (i, 0)),
            pl.BlockSpec((BP, 2 * D), lambda i: (0, 0)),
            pl.BlockSpec((BP, 1), lambda i: (0, 0)),
            pl.BlockSpec((1, D), lambda i: (0, 0)),
            pl.BlockSpec((1, D), lambda i: (0, 0)),
            pl.BlockSpec((1, D), lambda i: (0, 0)),
        ],
        out_specs=pl.BlockSpec((NBLK, D), lambda i: (i, 0)),
        out_shape=jax.ShapeDtypeStruct((NPAD, D), jnp.float32),
    )(hpre, bat2, stats, cnt2, g, b, a)
    return h2


def kernel(x, src_map, code_mask, batch, e_base_child, e_base_father,
           e_dfg_next, e_dfg_prev, e_code_next, e_code_prev, emb_table,
           Wl, bl, Wr, gamma, beta, alpha):
    edges = [e_base_child, e_base_father, e_dfg_next, e_dfg_prev,
             e_code_next, e_code_prev]
    i32 = jnp.int32

    sps, locs = [], []
    blk_lo = (jnp.arange(NBC, dtype=i32) * NBLK)
    ke_ar = jnp.arange(KE, dtype=i32)[None, :]
    for e in edges:
        src = e[0].astype(i32)
        dst = e[1].astype(i32)
        order = jnp.argsort(dst)
        ds = dst[order]
        ss = src[order]
        lo = jnp.searchsorted(ds, blk_lo).astype(i32)
        hi = jnp.searchsorted(ds, blk_lo + NBLK).astype(i32)
        pos = lo[:, None] + ke_ar
        valid = pos < hi[:, None]
        posc = jnp.minimum(pos, ds.shape[0] - 1)
        sps.append(jnp.where(valid, ss[posc], 0))
        locs.append(jnp.where(valid, ds[posc] - blk_lo[:, None], NBLK))
    sp = jnp.stack(sps, axis=1)
    loc = jnp.stack(locs, axis=1)[..., None]
    msg_idx = sp.reshape(-1)

    bat_pad = jnp.concatenate([batch.astype(i32),
                               jnp.full((NPAD - N,), BP - 1, i32)])
    bat2 = bat_pad[:, None]
    counts = jnp.bincount(batch, length=B).astype(jnp.float32)
    cnt2 = jnp.concatenate([jnp.maximum(counts, 1.0),
                            jnp.ones((BP - B,), jnp.float32)])[:, None]
    bsum = jnp.sum(bl, axis=1)

    emb_pad = jnp.concatenate([emb_table, jnp.zeros((8, D), jnp.float32)])
    xi = jnp.concatenate([x.astype(i32), jnp.full((NPAD - N,), V, i32)])
    h = _sc_gather(emb_pad, xi)

    for l in range(L):
        msgs = _sc_gather(h, msg_idx).reshape(NBC, R, KE, D)
        h = _run_layer(h, msgs, loc, bat2, cnt2, Wl[l], Wr[l],
                       bsum[l][None, :], gamma[l][None, :],
                       beta[l][None, :], alpha[l][None, :])

    h_fp = jnp.concatenate([h, jnp.zeros((8, D), jnp.float32)])
    starts = (jnp.cumsum(counts.astype(i32)) - counts.astype(i32))
    p_ar = jnp.arange(GMAX, dtype=i32)[None, :]
    ge_idx = starts[:, None] + p_ar
    ge_idx = jnp.where(p_ar < counts.astype(i32)[:, None], ge_idx, ZR)
    graph_enc = _sc_gather(h_fp, ge_idx.reshape(-1)).reshape(B, GMAX, D)

    m = code_mask.astype(i32)
    mcnt = jax.ops.segment_sum(m, batch, num_segments=B)
    mstart = jnp.cumsum(mcnt) - mcnt
    mpos = jnp.cumsum(m) - m - mstart[batch]
    validc = code_mask & (mpos < CMAX)
    slot = jnp.where(validc, batch * CMAX + mpos, B * CMAX)
    src_of_slot = jnp.full((B * CMAX + 1,), ZR, i32)
    src_of_slot = src_of_slot.at[slot].set(jnp.arange(N, dtype=i32))
    src_of_slot = src_of_slot[:B * CMAX]
    graph_code_enc = _sc_gather(h_fp, src_of_slot).reshape(B, CMAX, D)

    smt = jnp.zeros((NPAD + 8, D), i32).at[:N, :].set(
        src_map.astype(i32)[:, None])
    code_src_map = _sc_gather(smt, src_of_slot)[:, 0].reshape(B, CMAX)

    return (graph_enc, graph_code_enc, code_src_map)
